# baseline (speedup 1.0000x reference)
import functools
import math

import jax
import jax.numpy as jnp
from jax import lax
from jax.experimental import pallas as pl
from jax.experimental.pallas import tpu as pltpu
from jax.experimental.pallas import tpu_sc as plsc

_VOCAB = 1000000
_DIM = 64
_B = 4096 * 200
_NW = 32
_IROW = 128
_G = 4
_CHUNK = _G * _IROW
_ROWS_PER_W = _B // _NW
_NCHUNK = _ROWS_PER_W // _CHUNK
_SCALE = math.sqrt(_DIM)

_mesh = plsc.VectorSubcoreMesh(core_axis_name="c", subcore_axis_name="s")


@functools.partial(
    pl.kernel,
    out_type=jax.ShapeDtypeStruct((_B, _DIM), jnp.float32),
    mesh=_mesh,
    compiler_params=pltpu.CompilerParams(use_tc_tiling_on_sc=False),
    scratch_types=[
        pltpu.VMEM((_G, _IROW), jnp.int32),
        pltpu.VMEM((_CHUNK, _DIM), jnp.float32),
        pltpu.SemaphoreType.DMA,
    ],
)
def _emb_lookup(idx_hbm, table_hbm, out_hbm, idx_v, rows_v, sem):
    wid = lax.axis_index("s") * 2 + lax.axis_index("c")
    row0 = wid * (_ROWS_PER_W // _IROW)

    def chunk_body(i, carry):
        irow = row0 + i * _G
        pltpu.sync_copy(idx_hbm.at[pl.ds(irow, _G)], idx_v)
        copies = [
            pltpu.async_copy(
                table_hbm.at[idx_v.at[j]],
                rows_v.at[pl.ds(j * _IROW, _IROW)],
                sem,
            )
            for j in range(_G)
        ]
        for c in copies:
            c.wait()

        def scale_body(r, carry2):
            for j in range(_DIM // 16):
                sl = pl.ds(j * 16, 16)
                rows_v[r, sl] = rows_v[r, sl] * _SCALE
            return carry2

        lax.fori_loop(0, _CHUNK, scale_body, 0, unroll=4)
        pltpu.sync_copy(rows_v, out_hbm.at[pl.ds(irow * _IROW, _CHUNK)])
        return carry

    lax.fori_loop(0, _NCHUNK, chunk_body, 0)


def kernel(x, emb_table):
    idx = x.reshape(_B // _IROW, _IROW).astype(jnp.int32)
    out = _emb_lookup(idx, emb_table)
    return out.reshape(x.shape[0], x.shape[1], _DIM)


# speedup vs baseline: 1.2919x; 1.2919x over previous
import functools
import math

import jax
import jax.numpy as jnp
from jax import lax
from jax.experimental import pallas as pl
from jax.experimental.pallas import tpu as pltpu
from jax.experimental.pallas import tpu_sc as plsc

_VOCAB = 1000000
_DIM = 64
_B = 4096 * 200
_NW = 32
_IROW = 128
_G = 4
_CHUNK = _G * _IROW
_ROWS_PER_W = _B // _NW
_NCHUNK = _ROWS_PER_W // _CHUNK
_SCALE = math.sqrt(_DIM)

_mesh = plsc.VectorSubcoreMesh(core_axis_name="c", subcore_axis_name="s")


@functools.partial(
    pl.kernel,
    out_type=jax.ShapeDtypeStruct((_B, 2 * _DIM), jnp.float32),
    mesh=_mesh,
    compiler_params=pltpu.CompilerParams(use_tc_tiling_on_sc=False),
    scratch_types=[
        pltpu.VMEM((_CHUNK,), jnp.int32),
        pltpu.VMEM((_CHUNK, _DIM), jnp.float32),
        pltpu.SemaphoreType.DMA,
    ],
)
def _emb_lookup(idx_hbm, table_hbm, out_hbm, idx_v, rows_v, sem):
    wid = lax.axis_index("s") * 2 + lax.axis_index("c")
    base = wid * _ROWS_PER_W

    def chunk_body(i, carry):
        row0 = base + i * _CHUNK
        pltpu.sync_copy(idx_hbm.at[pl.ds(row0, _CHUNK)], idx_v)
        copies = [
            pltpu.async_copy(
                table_hbm.at[idx_v.at[pl.ds(j * _IROW, _IROW)]],
                rows_v.at[pl.ds(j * _IROW, _IROW)],
                sem,
            )
            for j in range(_G)
        ]
        for c in copies:
            c.wait()

        def scale_body(r):
            for j in range(_DIM // 16):
                sl = pl.ds(j * 16, 16)
                rows_v[r, sl] = rows_v[r, sl] * _SCALE

        plsc.parallel_loop(0, _CHUNK, 1, unroll=8)(scale_body)
        pltpu.sync_copy(
            rows_v, out_hbm.at[pl.ds(row0, _CHUNK), pl.ds(0, _DIM)]
        )
        return carry

    lax.fori_loop(0, _NCHUNK, chunk_body, 0)


def kernel(x, emb_table):
    idx = x.reshape(_B).astype(jnp.int32)
    out = _emb_lookup(idx, emb_table)
    return out[:, :_DIM].reshape(x.shape[0], x.shape[1], _DIM)
